# Initial kernel scaffold; baseline (speedup 1.0000x reference)
import functools

import jax
import jax.numpy as jnp
from jax import lax
from jax.experimental import pallas as pl
from jax.experimental.pallas import tpu as pltpu
from jax.experimental.pallas import tpu_sc as plsc

N = 10000
E = 320000
D = 128

NC = 2
NS = 16
NW = NC * NS

C = 128
N_PAD = 10240
E_PAD = 327680
EPT = E_PAD // NW
CHUNKS_MSG = EPT // C
EPS = E_PAD // NS
CHUNKS_DEG = EPS // C
ROWS_PER_SUB = N_PAD // NS

_mesh = plsc.VectorSubcoreMesh(core_axis_name="c", subcore_axis_name="s")


def _sc_degree_body(ei_hbm, zeros16_hbm, ones_hbm, deg_hbm, idx_v, ones_v, dacc, _):
    cid = lax.axis_index("c")
    sid = lax.axis_index("s")
    pltpu.sync_copy(zeros16_hbm.at[pl.ds(sid * ROWS_PER_SUB, ROWS_PER_SUB)],
                    dacc.at[pl.ds(sid * ROWS_PER_SUB, ROWS_PER_SUB)])
    pltpu.sync_copy(ones_hbm, ones_v)
    plsc.subcore_barrier()

    base = sid * EPS

    def chunk(j, carry):
        off = base + j * C
        pltpu.sync_copy(ei_hbm.at[cid, pl.ds(off, C)], idx_v)
        pltpu.sync_copy(ones_v, dacc.at[idx_v], add=True)
        return carry

    lax.fori_loop(0, CHUNKS_DEG, chunk, 0)
    plsc.subcore_barrier()
    pltpu.sync_copy(dacc.at[pl.ds(sid * ROWS_PER_SUB, ROWS_PER_SUB)],
                    deg_hbm.at[cid, pl.ds(sid * ROWS_PER_SUB, ROWS_PER_SUB)])


_sc_degree = pl.kernel(
    _sc_degree_body,
    out_type=jax.ShapeDtypeStruct((NC, N_PAD, 16), jnp.float32),
    mesh=_mesh,
    scratch_types=[
        pltpu.VMEM((C,), jnp.int32),
        pltpu.VMEM((C, 16), jnp.float32),
        pltpu.VMEM_SHARED((N_PAD, 16), jnp.float32),
        pltpu.SemaphoreType.DMA,
    ],
)


def _sc_msgpass_body(h_hbm, ei_hbm, zeros_hbm, out_hbm, src_v, dst_v, rows_v, acc, sem):
    cid = lax.axis_index("c")
    sid = lax.axis_index("s")
    wid = sid * NC + cid
    pltpu.sync_copy(zeros_hbm.at[pl.ds(sid * ROWS_PER_SUB, ROWS_PER_SUB)],
                    acc.at[pl.ds(sid * ROWS_PER_SUB, ROWS_PER_SUB)])
    plsc.subcore_barrier()

    base = wid * EPT

    def chunk(j, carry):
        off = base + j * C
        pltpu.sync_copy(ei_hbm.at[0, pl.ds(off, C)], src_v)
        pltpu.sync_copy(ei_hbm.at[1, pl.ds(off, C)], dst_v)
        pltpu.async_copy(h_hbm.at[src_v], rows_v, sem).wait()
        pltpu.sync_copy(rows_v, acc.at[dst_v], add=True)
        return carry

    lax.fori_loop(0, CHUNKS_MSG, chunk, 0)
    plsc.subcore_barrier()
    pltpu.sync_copy(acc.at[pl.ds(sid * ROWS_PER_SUB, ROWS_PER_SUB)],
                    out_hbm.at[cid, pl.ds(sid * ROWS_PER_SUB, ROWS_PER_SUB)])


_sc_msgpass = pl.kernel(
    _sc_msgpass_body,
    out_type=jax.ShapeDtypeStruct((NC, N_PAD, D), jnp.float32),
    mesh=_mesh,
    scratch_types=[
        pltpu.VMEM((C,), jnp.int32),
        pltpu.VMEM((C,), jnp.int32),
        pltpu.VMEM((C, D), jnp.float32),
        pltpu.VMEM_SHARED((N_PAD, D), jnp.float32),
        pltpu.SemaphoreType.DMA,
    ],
)

_BLK = 1024
_GRID = N_PAD // _BLK


def _tc_prescale_body(x_ref, deg_ref, o_ref):
    dout = lax.rsqrt(jnp.maximum(deg_ref[0, :, 0:1], 1.0))
    o_ref[...] = x_ref[...] * dout


_tc_prescale = pl.pallas_call(
    _tc_prescale_body,
    grid=(_GRID,),
    in_specs=[
        pl.BlockSpec((_BLK, D), lambda i: (i, 0)),
        pl.BlockSpec((NC, _BLK, 16), lambda i: (0, i, 0)),
    ],
    out_specs=pl.BlockSpec((_BLK, D), lambda i: (i, 0)),
    out_shape=jax.ShapeDtypeStruct((N_PAD, D), jnp.float32),
)


def _tc_layer_body(p_ref, deg_ref, w_ref, b_ref, o_ref, *, last):
    din = lax.rsqrt(jnp.maximum(deg_ref[1, :, 0:1], 1.0))
    m = (p_ref[0] + p_ref[1]) * din
    y = jnp.dot(m, w_ref[...], preferred_element_type=jnp.float32) + b_ref[...]
    if last:
        o_ref[...] = jnp.max(y, axis=1, keepdims=True)
    else:
        dout = lax.rsqrt(jnp.maximum(deg_ref[0, :, 0:1], 1.0))
        o_ref[...] = jnp.maximum(y, 0.0) * dout


def _make_tc_layer(last):
    return pl.pallas_call(
        functools.partial(_tc_layer_body, last=last),
        grid=(_GRID,),
        in_specs=[
            pl.BlockSpec((NC, _BLK, D), lambda i: (0, i, 0)),
            pl.BlockSpec((NC, _BLK, 16), lambda i: (0, i, 0)),
            pl.BlockSpec((D, D), lambda i: (0, 0)),
            pl.BlockSpec((1, D), lambda i: (0, 0)),
        ],
        out_specs=pl.BlockSpec((_BLK, 1 if last else D), lambda i: (i, 0)),
        out_shape=jax.ShapeDtypeStruct((N_PAD, 1 if last else D), jnp.float32),
    )


_tc_layer_mid = _make_tc_layer(last=False)
_tc_layer_last = _make_tc_layer(last=True)


def kernel(x, edge_index, W1, b1, W2, b2, W3, b3, W4, b4):
    ei = jnp.concatenate(
        [edge_index, jnp.full((2, E_PAD - E), N, dtype=jnp.int32)], axis=1)
    x_pad = jnp.concatenate(
        [x, jnp.zeros((N_PAD - N, D), dtype=jnp.float32)], axis=0)
    zeros128 = jnp.zeros((N_PAD, D), dtype=jnp.float32)
    zeros16 = jnp.zeros((N_PAD, 16), dtype=jnp.float32)
    ones16 = jnp.ones((C, 16), dtype=jnp.float32)

    deg = _sc_degree(ei, zeros16, ones16)
    h = _tc_prescale(x_pad, deg)
    for w, b, last in ((W1, b1, False), (W2, b2, False), (W3, b3, False),
                       (W4, b4, True)):
        p = _sc_msgpass(h, ei, zeros128)
        layer = _tc_layer_last if last else _tc_layer_mid
        h = layer(p, deg, w, b.reshape(1, D))
    return h[:N, 0]


# speedup vs baseline: 1.3181x; 1.3181x over previous
import functools

import jax
import jax.numpy as jnp
from jax import lax
from jax.experimental import pallas as pl
from jax.experimental.pallas import tpu as pltpu
from jax.experimental.pallas import tpu_sc as plsc

N = 10000
E = 320000
D = 128

NC = 2
NS = 16
NW = NC * NS

C = 128
N_PAD = 10240
E_PAD = 327680
EPT = E_PAD // NW
CHUNKS_MSG = EPT // C
ROWS_PER_SUB = N_PAD // NS

_mesh = plsc.VectorSubcoreMesh(core_axis_name="c", subcore_axis_name="s")


def _sc_msgpass_body(h_hbm, ei_hbm, zeros_hbm, out_hbm, src_v, dst_v, rows_v, acc, sem):
    cid = lax.axis_index("c")
    sid = lax.axis_index("s")
    wid = sid * NC + cid
    pltpu.sync_copy(zeros_hbm.at[pl.ds(sid * ROWS_PER_SUB, ROWS_PER_SUB)],
                    acc.at[pl.ds(sid * ROWS_PER_SUB, ROWS_PER_SUB)])
    plsc.subcore_barrier()

    base = wid * EPT

    def chunk(j, carry):
        off = base + j * C
        pltpu.sync_copy(ei_hbm.at[0, pl.ds(off, C)], src_v)
        pltpu.sync_copy(ei_hbm.at[1, pl.ds(off, C)], dst_v)
        pltpu.async_copy(h_hbm.at[src_v], rows_v, sem).wait()
        pltpu.sync_copy(rows_v, acc.at[dst_v], add=True)
        return carry

    lax.fori_loop(0, CHUNKS_MSG, chunk, 0)
    plsc.subcore_barrier()
    pltpu.sync_copy(acc.at[pl.ds(sid * ROWS_PER_SUB, ROWS_PER_SUB)],
                    out_hbm.at[cid, pl.ds(sid * ROWS_PER_SUB, ROWS_PER_SUB)])


_sc_msgpass = pl.kernel(
    _sc_msgpass_body,
    out_type=jax.ShapeDtypeStruct((NC, N_PAD, D), jnp.float32),
    mesh=_mesh,
    scratch_types=[
        pltpu.VMEM((C,), jnp.int32),
        pltpu.VMEM((C,), jnp.int32),
        pltpu.VMEM((C, D), jnp.float32),
        pltpu.VMEM_SHARED((N_PAD, D), jnp.float32),
        pltpu.SemaphoreType.DMA,
    ],
)

_BLK = 1024
_GRID = N_PAD // _BLK


def _tc_prescale_body(x_ref, dol_ref, o_ref):
    cnt = dol_ref[0, :, 0:1] + dol_ref[1, :, 0:1]
    o_ref[...] = x_ref[...] * lax.rsqrt(jnp.maximum(cnt, 1.0))


_tc_prescale = pl.pallas_call(
    _tc_prescale_body,
    grid=(_GRID,),
    in_specs=[
        pl.BlockSpec((_BLK, D), lambda i: (i, 0)),
        pl.BlockSpec((NC, _BLK, D), lambda i: (0, i, 0)),
    ],
    out_specs=pl.BlockSpec((_BLK, D), lambda i: (i, 0)),
    out_shape=jax.ShapeDtypeStruct((N_PAD, D), jnp.float32),
)


def _tc_layer_body(p_ref, dil_ref, dol_ref, w_ref, b_ref, o_ref, *, last):
    cin = dil_ref[0, :, 0:1] + dil_ref[1, :, 0:1]
    m = (p_ref[0] + p_ref[1]) * lax.rsqrt(jnp.maximum(cin, 1.0))
    y = jnp.dot(m, w_ref[...], preferred_element_type=jnp.float32) + b_ref[...]
    if last:
        o_ref[...] = jnp.max(y, axis=1, keepdims=True)
    else:
        cout = dol_ref[0, :, 0:1] + dol_ref[1, :, 0:1]
        o_ref[...] = jnp.maximum(y, 0.0) * lax.rsqrt(jnp.maximum(cout, 1.0))


def _make_tc_layer(last):
    return pl.pallas_call(
        functools.partial(_tc_layer_body, last=last),
        grid=(_GRID,),
        in_specs=[
            pl.BlockSpec((NC, _BLK, D), lambda i: (0, i, 0)),
            pl.BlockSpec((NC, _BLK, D), lambda i: (0, i, 0)),
            pl.BlockSpec((NC, _BLK, D), lambda i: (0, i, 0)),
            pl.BlockSpec((D, D), lambda i: (0, 0)),
            pl.BlockSpec((1, D), lambda i: (0, 0)),
        ],
        out_specs=pl.BlockSpec((_BLK, 1 if last else D), lambda i: (i, 0)),
        out_shape=jax.ShapeDtypeStruct((N_PAD, 1 if last else D), jnp.float32),
    )


_tc_layer_mid = _make_tc_layer(last=False)
_tc_layer_last = _make_tc_layer(last=True)


def kernel(x, edge_index, W1, b1, W2, b2, W3, b3, W4, b4):
    ei = jnp.concatenate(
        [edge_index, jnp.full((2, E_PAD - E), N, dtype=jnp.int32)], axis=1)
    ei_flip = ei[::-1]
    x_pad = jnp.concatenate(
        [x, jnp.zeros((N_PAD - N, D), dtype=jnp.float32)], axis=0)
    zeros128 = jnp.zeros((N_PAD, D), dtype=jnp.float32)
    ones128 = jnp.ones((N_PAD, D), dtype=jnp.float32)

    dil = _sc_msgpass(ones128, ei, zeros128)
    dol = _sc_msgpass(ones128, ei_flip, zeros128)
    h = _tc_prescale(x_pad, dol)
    for w, b, last in ((W1, b1, False), (W2, b2, False), (W3, b3, False),
                       (W4, b4, True)):
        p = _sc_msgpass(h, ei, zeros128)
        layer = _tc_layer_last if last else _tc_layer_mid
        h = layer(p, dil, dol, w, b.reshape(1, D))
    return h[:N, 0]


# speedup vs baseline: 1.5695x; 1.1908x over previous
import functools

import jax
import jax.numpy as jnp
from jax import lax
from jax.experimental import pallas as pl
from jax.experimental.pallas import tpu as pltpu
from jax.experimental.pallas import tpu_sc as plsc

N = 10000
E = 320000
D = 128

NC = 2
NS = 16
NW = NC * NS

C = 128
N_PAD = 10240
E_PAD = 327680
EPT = E_PAD // NW
CHUNKS_MSG = EPT // C
ROWS_PER_SUB = N_PAD // NS

_mesh = plsc.VectorSubcoreMesh(core_axis_name="c", subcore_axis_name="s")


def _sc_msgpass_body(h_hbm, ei_hbm, zeros_hbm, out_hbm, srcs, dsts, rows, acc,
                     semg0, semg1):
    cid = lax.axis_index("c")
    sid = lax.axis_index("s")
    wid = sid * NC + cid
    pltpu.sync_copy(zeros_hbm.at[pl.ds(sid * ROWS_PER_SUB, ROWS_PER_SUB)],
                    acc.at[pl.ds(sid * ROWS_PER_SUB, ROWS_PER_SUB)])
    plsc.subcore_barrier()

    base = wid * EPT
    sems = (semg0, semg1)

    def load_idx(j, b):
        off = base + j * C
        pltpu.sync_copy(ei_hbm.at[0, pl.ds(off, C)], srcs.at[b])
        pltpu.sync_copy(ei_hbm.at[1, pl.ds(off, C)], dsts.at[b])

    def start_gather(b):
        pltpu.async_copy(h_hbm.at[srcs.at[b]], rows.at[b], sems[b])

    def wait_gather(b):
        pltpu.make_async_copy(h_hbm.at[srcs.at[b]], rows.at[b], sems[b]).wait()

    def scatter(b):
        pltpu.sync_copy(rows.at[b], acc.at[dsts.at[b]], add=True)

    load_idx(0, 0)
    start_gather(0)

    def pair(jj, carry):
        j0 = 2 * jj
        load_idx(j0 + 1, 1)
        start_gather(1)
        wait_gather(0)
        scatter(0)

        @pl.when(jj + 1 < CHUNKS_MSG // 2)
        def _():
            load_idx(j0 + 2, 0)
            start_gather(0)

        wait_gather(1)
        scatter(1)
        return carry

    lax.fori_loop(0, CHUNKS_MSG // 2, pair, 0)
    plsc.subcore_barrier()
    pltpu.sync_copy(acc.at[pl.ds(sid * ROWS_PER_SUB, ROWS_PER_SUB)],
                    out_hbm.at[cid, pl.ds(sid * ROWS_PER_SUB, ROWS_PER_SUB)])


_sc_msgpass = pl.kernel(
    _sc_msgpass_body,
    out_type=jax.ShapeDtypeStruct((NC, N_PAD, D), jnp.float32),
    mesh=_mesh,
    scratch_types=[
        pltpu.VMEM((2, C), jnp.int32),
        pltpu.VMEM((2, C), jnp.int32),
        pltpu.VMEM((2, C, D), jnp.float32),
        pltpu.VMEM_SHARED((N_PAD, D), jnp.float32),
        pltpu.SemaphoreType.DMA,
        pltpu.SemaphoreType.DMA,
    ],
)

_BLK = 1024
_GRID = N_PAD // _BLK


def _tc_prescale_body(x_ref, dol_ref, o_ref):
    cnt = dol_ref[0, :, 0:1] + dol_ref[1, :, 0:1]
    o_ref[...] = x_ref[...] * lax.rsqrt(jnp.maximum(cnt, 1.0))


_tc_prescale = pl.pallas_call(
    _tc_prescale_body,
    grid=(_GRID,),
    in_specs=[
        pl.BlockSpec((_BLK, D), lambda i: (i, 0)),
        pl.BlockSpec((NC, _BLK, D), lambda i: (0, i, 0)),
    ],
    out_specs=pl.BlockSpec((_BLK, D), lambda i: (i, 0)),
    out_shape=jax.ShapeDtypeStruct((N_PAD, D), jnp.float32),
)


def _tc_layer_body(p_ref, dil_ref, dol_ref, w_ref, b_ref, o_ref, *, last):
    cin = dil_ref[0, :, 0:1] + dil_ref[1, :, 0:1]
    m = (p_ref[0] + p_ref[1]) * lax.rsqrt(jnp.maximum(cin, 1.0))
    y = jnp.dot(m, w_ref[...], preferred_element_type=jnp.float32) + b_ref[...]
    if last:
        o_ref[...] = jnp.max(y, axis=1, keepdims=True)
    else:
        cout = dol_ref[0, :, 0:1] + dol_ref[1, :, 0:1]
        o_ref[...] = jnp.maximum(y, 0.0) * lax.rsqrt(jnp.maximum(cout, 1.0))


def _make_tc_layer(last):
    return pl.pallas_call(
        functools.partial(_tc_layer_body, last=last),
        grid=(_GRID,),
        in_specs=[
            pl.BlockSpec((NC, _BLK, D), lambda i: (0, i, 0)),
            pl.BlockSpec((NC, _BLK, D), lambda i: (0, i, 0)),
            pl.BlockSpec((NC, _BLK, D), lambda i: (0, i, 0)),
            pl.BlockSpec((D, D), lambda i: (0, 0)),
            pl.BlockSpec((1, D), lambda i: (0, 0)),
        ],
        out_specs=pl.BlockSpec((_BLK, 1 if last else D), lambda i: (i, 0)),
        out_shape=jax.ShapeDtypeStruct((N_PAD, 1 if last else D), jnp.float32),
    )


_tc_layer_mid = _make_tc_layer(last=False)
_tc_layer_last = _make_tc_layer(last=True)


def kernel(x, edge_index, W1, b1, W2, b2, W3, b3, W4, b4):
    ei = jnp.concatenate(
        [edge_index, jnp.full((2, E_PAD - E), N, dtype=jnp.int32)], axis=1)
    ei_flip = ei[::-1]
    x_pad = jnp.concatenate(
        [x, jnp.zeros((N_PAD - N, D), dtype=jnp.float32)], axis=0)
    zeros128 = jnp.zeros((N_PAD, D), dtype=jnp.float32)
    ones128 = jnp.ones((N_PAD, D), dtype=jnp.float32)

    dil = _sc_msgpass(ones128, ei, zeros128)
    dol = _sc_msgpass(ones128, ei_flip, zeros128)
    h = _tc_prescale(x_pad, dol)
    for w, b, last in ((W1, b1, False), (W2, b2, False), (W3, b3, False),
                       (W4, b4, True)):
        p = _sc_msgpass(h, ei, zeros128)
        layer = _tc_layer_last if last else _tc_layer_mid
        h = layer(p, dil, dol, w, b.reshape(1, D))
    return h[:N, 0]


# speedup vs baseline: 2.4835x; 1.5823x over previous
import functools

import jax
import jax.numpy as jnp
from jax import lax
from jax.experimental import pallas as pl
from jax.experimental.pallas import tpu as pltpu
from jax.experimental.pallas import tpu_sc as plsc

N = 10000
E = 320000
D = 128

NC = 2
NS = 16
NW = NC * NS

C = 128
N_PAD = 10240
E_PAD = 327680
EPT = E_PAD // NW
CHUNKS_MSG = EPT // C
ROWS_PER_SUB = N_PAD // NS

_mesh = plsc.VectorSubcoreMesh(core_axis_name="c", subcore_axis_name="s")


def _sc_msgpass_body(h_hbm, ei_hbm, zeros_hbm, out_hbm, srcs, dsts, rows, acc,
                     semg0, semg1, *, srow=0, drow=1):
    cid = lax.axis_index("c")
    sid = lax.axis_index("s")
    wid = sid * NC + cid
    pltpu.sync_copy(zeros_hbm.at[pl.ds(sid * ROWS_PER_SUB, ROWS_PER_SUB)],
                    acc.at[pl.ds(sid * ROWS_PER_SUB, ROWS_PER_SUB)])
    plsc.subcore_barrier()

    base = wid * EPT
    sems = (semg0, semg1)

    def load_idx(j, b):
        off = base + j * C
        pltpu.sync_copy(ei_hbm.at[srow, pl.ds(off, C)], srcs.at[b])
        pltpu.sync_copy(ei_hbm.at[drow, pl.ds(off, C)], dsts.at[b])

    def start_gather(b):
        pltpu.async_copy(h_hbm.at[srcs.at[b]], rows.at[b], sems[b])

    def wait_gather(b):
        pltpu.make_async_copy(h_hbm.at[srcs.at[b]], rows.at[b], sems[b]).wait()

    def scatter(b):
        pltpu.sync_copy(rows.at[b], acc.at[dsts.at[b]], add=True)

    load_idx(0, 0)
    start_gather(0)

    def pair(jj, carry):
        j0 = 2 * jj
        load_idx(j0 + 1, 1)
        start_gather(1)
        wait_gather(0)
        scatter(0)

        @pl.when(jj + 1 < CHUNKS_MSG // 2)
        def _():
            load_idx(j0 + 2, 0)
            start_gather(0)

        wait_gather(1)
        scatter(1)
        return carry

    lax.fori_loop(0, CHUNKS_MSG // 2, pair, 0)
    plsc.subcore_barrier()
    pltpu.sync_copy(acc.at[pl.ds(sid * ROWS_PER_SUB, ROWS_PER_SUB)],
                    out_hbm.at[cid, pl.ds(sid * ROWS_PER_SUB, ROWS_PER_SUB)])


def _make_sc_msgpass(srow, drow):
    return pl.kernel(
        functools.partial(_sc_msgpass_body, srow=srow, drow=drow),
        out_type=jax.ShapeDtypeStruct((NC, N_PAD, D), jnp.float32),
        mesh=_mesh,
        scratch_types=[
            pltpu.VMEM((2, C), jnp.int32),
            pltpu.VMEM((2, C), jnp.int32),
            pltpu.VMEM((2, C, D), jnp.float32),
            pltpu.VMEM_SHARED((N_PAD, D), jnp.float32),
            pltpu.SemaphoreType.DMA,
            pltpu.SemaphoreType.DMA,
        ],
    )


_sc_msgpass = _make_sc_msgpass(0, 1)
_sc_msgpass_rev = _make_sc_msgpass(1, 0)

_BLK = 1024
_GRID = N_PAD // _BLK


def _tc_prescale_body(x_ref, dol_ref, o_ref):
    cnt = dol_ref[0, :, 0:1] + dol_ref[1, :, 0:1]
    o_ref[...] = x_ref[...] * lax.rsqrt(jnp.maximum(cnt, 1.0))


_tc_prescale = pl.pallas_call(
    _tc_prescale_body,
    grid=(_GRID,),
    in_specs=[
        pl.BlockSpec((_BLK, D), lambda i: (i, 0)),
        pl.BlockSpec((NC, _BLK, D), lambda i: (0, i, 0)),
    ],
    out_specs=pl.BlockSpec((_BLK, D), lambda i: (i, 0)),
    out_shape=jax.ShapeDtypeStruct((N_PAD, D), jnp.float32),
)


def _tc_layer_body(p_ref, dil_ref, dol_ref, w_ref, b_ref, o_ref, *, last):
    cin = dil_ref[0, :, 0:1] + dil_ref[1, :, 0:1]
    m = (p_ref[0] + p_ref[1]) * lax.rsqrt(jnp.maximum(cin, 1.0))
    y = jnp.dot(m, w_ref[...], preferred_element_type=jnp.float32) + b_ref[...]
    if last:
        o_ref[...] = jnp.max(y, axis=1, keepdims=True)
    else:
        cout = dol_ref[0, :, 0:1] + dol_ref[1, :, 0:1]
        o_ref[...] = jnp.maximum(y, 0.0) * lax.rsqrt(jnp.maximum(cout, 1.0))


def _make_tc_layer(last):
    return pl.pallas_call(
        functools.partial(_tc_layer_body, last=last),
        grid=(_GRID,),
        in_specs=[
            pl.BlockSpec((NC, _BLK, D), lambda i: (0, i, 0)),
            pl.BlockSpec((NC, _BLK, D), lambda i: (0, i, 0)),
            pl.BlockSpec((NC, _BLK, D), lambda i: (0, i, 0)),
            pl.BlockSpec((D, D), lambda i: (0, 0)),
            pl.BlockSpec((1, D), lambda i: (0, 0)),
        ],
        out_specs=pl.BlockSpec((_BLK, 1 if last else D), lambda i: (i, 0)),
        out_shape=jax.ShapeDtypeStruct((N_PAD, 1 if last else D), jnp.float32),
    )


_tc_layer_mid = _make_tc_layer(last=False)
_tc_layer_last = _make_tc_layer(last=True)


def kernel(x, edge_index, W1, b1, W2, b2, W3, b3, W4, b4):
    ei = jnp.concatenate(
        [edge_index, jnp.full((2, E_PAD - E), N, dtype=jnp.int32)], axis=1)
    x_pad = jnp.concatenate(
        [x, jnp.zeros((N_PAD - N, D), dtype=jnp.float32)], axis=0)
    zeros128 = jnp.zeros((N_PAD, D), dtype=jnp.float32)
    ones128 = jnp.ones((N_PAD, D), dtype=jnp.float32)

    dil = _sc_msgpass(ones128, ei, zeros128)
    dol = _sc_msgpass_rev(ones128, ei, zeros128)
    h = _tc_prescale(x_pad, dol)
    for w, b, last in ((W1, b1, False), (W2, b2, False), (W3, b3, False),
                       (W4, b4, True)):
        p = _sc_msgpass(h, ei, zeros128)
        layer = _tc_layer_last if last else _tc_layer_mid
        h = layer(p, dil, dol, w, b.reshape(1, D))
    return h[:N, 0]


# speedup vs baseline: 3.1625x; 1.2734x over previous
import functools

import jax
import jax.numpy as jnp
from jax import lax
from jax.experimental import pallas as pl
from jax.experimental.pallas import tpu as pltpu
from jax.experimental.pallas import tpu_sc as plsc

N = 10000
E = 320000
D = 128

NC = 2
NS = 16
NW = NC * NS

C = 128
N_PAD = 10240
E_PAD = 327680
EPT = E_PAD // NW
CHUNKS_MSG = EPT // C
ROWS_PER_SUB = N_PAD // NS

_mesh = plsc.VectorSubcoreMesh(core_axis_name="c", subcore_axis_name="s")

EPS = E_PAD // NS
CHUNKS_DEG = EPS // C


def _sc_degree_body(ei_hbm, out_hbm, idx_v, ones_v, zstage, dacc):
    cid = lax.axis_index("c")
    sid = lax.axis_index("s")

    def fill_ones(i, carry):
        for k in range(8):
            ones_v[i, pl.ds(16 * k, 16)] = jnp.full((16,), 1.0, jnp.float32)
        return carry

    lax.fori_loop(0, C, fill_ones, 0)

    def fill_zero(i, carry):
        for k in range(8):
            zstage[i, pl.ds(16 * k, 16)] = jnp.zeros((16,), jnp.float32)
        return carry

    lax.fori_loop(0, 64, fill_zero, 0)

    def zero_chunk(cch, carry):
        pltpu.sync_copy(zstage,
                        dacc.at[pl.ds(sid * ROWS_PER_SUB + cch * 64, 64)])
        return carry

    lax.fori_loop(0, ROWS_PER_SUB // 64, zero_chunk, 0)
    plsc.subcore_barrier()

    def edge_loop(row):
        def chunk(j, carry):
            off = sid * EPS + j * C
            pltpu.sync_copy(ei_hbm.at[row, pl.ds(off, C)], idx_v)
            pltpu.sync_copy(ones_v, dacc.at[idx_v], add=True)
            return carry

        lax.fori_loop(0, CHUNKS_DEG, chunk, 0)

    @pl.when(cid == 0)
    def _():
        edge_loop(0)

    @pl.when(cid == 1)
    def _():
        edge_loop(1)

    plsc.subcore_barrier()
    pltpu.sync_copy(dacc.at[pl.ds(sid * ROWS_PER_SUB, ROWS_PER_SUB)],
                    out_hbm.at[cid, pl.ds(sid * ROWS_PER_SUB, ROWS_PER_SUB)])


_sc_degree = pl.kernel(
    _sc_degree_body,
    out_type=jax.ShapeDtypeStruct((NC, N_PAD, D), jnp.float32),
    mesh=_mesh,
    scratch_types=[
        pltpu.VMEM((C,), jnp.int32),
        pltpu.VMEM((C, D), jnp.float32),
        pltpu.VMEM((64, D), jnp.float32),
        pltpu.VMEM_SHARED((N_PAD, D), jnp.float32),
    ],
)


def _sc_msgpass_body(h_hbm, ei_hbm, zeros_hbm, out_hbm, srcs, dsts, rows, acc,
                     semg0, semg1, *, srow=0, drow=1):
    cid = lax.axis_index("c")
    sid = lax.axis_index("s")
    wid = sid * NC + cid
    pltpu.sync_copy(zeros_hbm.at[pl.ds(sid * ROWS_PER_SUB, ROWS_PER_SUB)],
                    acc.at[pl.ds(sid * ROWS_PER_SUB, ROWS_PER_SUB)])
    plsc.subcore_barrier()

    base = wid * EPT
    sems = (semg0, semg1)

    def load_idx(j, b):
        off = base + j * C
        pltpu.sync_copy(ei_hbm.at[srow, pl.ds(off, C)], srcs.at[b])
        pltpu.sync_copy(ei_hbm.at[drow, pl.ds(off, C)], dsts.at[b])

    def start_gather(b):
        pltpu.async_copy(h_hbm.at[srcs.at[b]], rows.at[b], sems[b])

    def wait_gather(b):
        pltpu.make_async_copy(h_hbm.at[srcs.at[b]], rows.at[b], sems[b]).wait()

    def scatter(b):
        pltpu.sync_copy(rows.at[b], acc.at[dsts.at[b]], add=True)

    load_idx(0, 0)
    start_gather(0)

    def pair(jj, carry):
        j0 = 2 * jj
        load_idx(j0 + 1, 1)
        start_gather(1)
        wait_gather(0)
        scatter(0)

        @pl.when(jj + 1 < CHUNKS_MSG // 2)
        def _():
            load_idx(j0 + 2, 0)
            start_gather(0)

        wait_gather(1)
        scatter(1)
        return carry

    lax.fori_loop(0, CHUNKS_MSG // 2, pair, 0)
    plsc.subcore_barrier()
    pltpu.sync_copy(acc.at[pl.ds(sid * ROWS_PER_SUB, ROWS_PER_SUB)],
                    out_hbm.at[cid, pl.ds(sid * ROWS_PER_SUB, ROWS_PER_SUB)])


def _make_sc_msgpass(srow, drow):
    return pl.kernel(
        functools.partial(_sc_msgpass_body, srow=srow, drow=drow),
        out_type=jax.ShapeDtypeStruct((NC, N_PAD, D), jnp.float32),
        mesh=_mesh,
        scratch_types=[
            pltpu.VMEM((2, C), jnp.int32),
            pltpu.VMEM((2, C), jnp.int32),
            pltpu.VMEM((2, C, D), jnp.float32),
            pltpu.VMEM_SHARED((N_PAD, D), jnp.float32),
            pltpu.SemaphoreType.DMA,
            pltpu.SemaphoreType.DMA,
        ],
    )


_sc_msgpass = _make_sc_msgpass(0, 1)

_BLK = 1024
_GRID = N_PAD // _BLK


def _tc_prescale_body(x_ref, deg_ref, o_ref):
    o_ref[...] = x_ref[...] * lax.rsqrt(jnp.maximum(deg_ref[0], 1.0))


_tc_prescale = pl.pallas_call(
    _tc_prescale_body,
    grid=(_GRID,),
    in_specs=[
        pl.BlockSpec((_BLK, D), lambda i: (i, 0)),
        pl.BlockSpec((NC, _BLK, D), lambda i: (0, i, 0)),
    ],
    out_specs=pl.BlockSpec((_BLK, D), lambda i: (i, 0)),
    out_shape=jax.ShapeDtypeStruct((N_PAD, D), jnp.float32),
)


def _tc_layer_body(p_ref, deg_ref, w_ref, b_ref, o_ref, *, last):
    m = (p_ref[0] + p_ref[1]) * lax.rsqrt(jnp.maximum(deg_ref[1], 1.0))
    y = jnp.dot(m, w_ref[...], preferred_element_type=jnp.float32) + b_ref[...]
    if last:
        o_ref[...] = jnp.max(y, axis=1, keepdims=True)
    else:
        o_ref[...] = jnp.maximum(y, 0.0) * lax.rsqrt(
            jnp.maximum(deg_ref[0], 1.0))


def _make_tc_layer(last):
    return pl.pallas_call(
        functools.partial(_tc_layer_body, last=last),
        grid=(_GRID,),
        in_specs=[
            pl.BlockSpec((NC, _BLK, D), lambda i: (0, i, 0)),
            pl.BlockSpec((NC, _BLK, D), lambda i: (0, i, 0)),
            pl.BlockSpec((D, D), lambda i: (0, 0)),
            pl.BlockSpec((1, D), lambda i: (0, 0)),
        ],
        out_specs=pl.BlockSpec((_BLK, 1 if last else D), lambda i: (i, 0)),
        out_shape=jax.ShapeDtypeStruct((N_PAD, 1 if last else D), jnp.float32),
    )


_tc_layer_mid = _make_tc_layer(last=False)
_tc_layer_last = _make_tc_layer(last=True)


def kernel(x, edge_index, W1, b1, W2, b2, W3, b3, W4, b4):
    ei = jnp.concatenate(
        [edge_index, jnp.full((2, E_PAD - E), N, dtype=jnp.int32)], axis=1)
    x_pad = jnp.concatenate(
        [x, jnp.zeros((N_PAD - N, D), dtype=jnp.float32)], axis=0)
    zeros128 = jnp.zeros((N_PAD, D), dtype=jnp.float32)

    deg = _sc_degree(ei)
    h = _tc_prescale(x_pad, deg)
    for w, b, last in ((W1, b1, False), (W2, b2, False), (W3, b3, False),
                       (W4, b4, True)):
        p = _sc_msgpass(h, ei, zeros128)
        layer = _tc_layer_last if last else _tc_layer_mid
        h = layer(p, deg, w, b.reshape(1, D))
    return h[:N, 0]
